# baseline (speedup 1.0000x reference)
import jax
import jax.numpy as jnp
from jax.experimental import pallas as pl

_TEMP = 0.1
_N = 1024
_M = 2 * _N
_D = 256
_BLK = 256


def _softplus(x):
    xc = jnp.minimum(x, 80.0)
    return jnp.log1p(jnp.exp(xc)) + jnp.maximum(x - 80.0, 0.0)


def _scl_kernel(lab_all_ref, key_all_ref, lab_col_ref, key_col_ref,
                frow_ref, fall_ref, ones_ref, out_ref):
    i = pl.program_id(0)
    frow = frow_ref[...]
    ones = ones_ref[...]

    def rsum(y):
        return jax.lax.dot_general(
            y, ones, dimension_numbers=(((1,), (0,)), ((), ())),
            preferred_element_type=jnp.float32)

    lg = jax.lax.dot_general(
        frow, fall_ref[...],
        dimension_numbers=(((1,), (1,)), ((), ())),
        preferred_element_type=jnp.float32)

    fm_lab = (lab_col_ref[...] == lab_all_ref[...]).astype(jnp.float32)
    fm_pair = (key_col_ref[...] == key_all_ref[...]).astype(jnp.float32)
    fm_lab_n = 1.0 - fm_lab
    fm_pair_n = 1.0 - fm_pair

    m_all = jnp.max(lg - fm_pair * 1e30, axis=1, keepdims=True)
    e = jnp.exp(jnp.minimum(lg - m_all, 80.0))
    s_id = rsum(e * fm_lab_n)
    s_cam = rsum(e * fm_pair_n)
    log_s_id = m_all + jnp.log(s_id)
    log_s_cam = m_all + jnp.log(s_cam)

    lg_diag = jnp.sum(frow * frow, axis=1, keepdims=True)

    cnt_id = rsum(fm_lab) - 1.0
    cnt_cam = rsum(fm_pair) - 1.0

    t_id = rsum(_softplus(log_s_id - lg) * fm_lab)
    t_cam = rsum(_softplus(log_s_cam - lg) * fm_pair)
    row_id = (t_id - _softplus(log_s_id - lg_diag)) / cnt_id
    row_cam = (t_cam - _softplus(log_s_cam - lg_diag)) / cnt_cam

    blk_sum = (jnp.sum(row_id + 0.5 * row_cam) * (1.0 / _M)).reshape(1, 1)

    @pl.when(i == 0)
    def _():
        out_ref[...] = jnp.zeros_like(out_ref)

    out_ref[...] += blk_sum


def kernel(features, label, camid):
    f = features.reshape(_M, _D) * jnp.sqrt(jnp.float32(1.0 / _TEMP))
    lab2 = jnp.repeat(label, 2)
    key2 = lab2 * 8 + jnp.repeat(camid, 2)
    lab_all = lab2.reshape(1, _M)
    key_all = key2.reshape(1, _M)
    lab_col = lab2.reshape(_M, 1)
    key_col = key2.reshape(_M, 1)
    ones = jnp.ones((_M, 1), jnp.float32)

    grid = _M // _BLK
    out = pl.pallas_call(
        _scl_kernel,
        grid=(grid,),
        in_specs=[
            pl.BlockSpec((1, _M), lambda i: (0, 0)),
            pl.BlockSpec((1, _M), lambda i: (0, 0)),
            pl.BlockSpec((_BLK, 1), lambda i: (i, 0)),
            pl.BlockSpec((_BLK, 1), lambda i: (i, 0)),
            pl.BlockSpec((_BLK, _D), lambda i: (i, 0)),
            pl.BlockSpec((_M, _D), lambda i: (0, 0)),
            pl.BlockSpec((_M, 1), lambda i: (0, 0)),
        ],
        out_specs=pl.BlockSpec((1, 1), lambda i: (0, 0)),
        out_shape=jax.ShapeDtypeStruct((1, 1), jnp.float32),
    )(lab_all, key_all, lab_col, key_col, f, f, ones)
    return out.reshape(())


# speedup vs baseline: 1.0950x; 1.0950x over previous
import jax
import jax.numpy as jnp
from jax.experimental import pallas as pl

_TEMP = 0.1
_N = 1024
_M = 2 * _N
_D = 256
_BLK = 512


def _softplus(x):
    return jnp.where(x > 80.0, x, jnp.log1p(jnp.exp(x)))


def _scl_kernel(lab_all_ref, key_all_ref, lab_col_ref, key_col_ref,
                frow_ref, fall_ref, ones_ref, out_ref):
    i = pl.program_id(0)
    frow = frow_ref[...]
    ones = ones_ref[...]

    def rsum(y):
        return jax.lax.dot_general(
            y, ones, dimension_numbers=(((1,), (0,)), ((), ())),
            preferred_element_type=jnp.float32)

    lg = jax.lax.dot_general(
        frow, fall_ref[...],
        dimension_numbers=(((1,), (1,)), ((), ())),
        preferred_element_type=jnp.float32)

    fm_lab = (lab_col_ref[...] == lab_all_ref[...]).astype(jnp.float32)
    fm_pair = (key_col_ref[...] == key_all_ref[...]).astype(jnp.float32)
    fm_lab_n = 1.0 - fm_lab
    fm_pair_n = 1.0 - fm_pair

    m_all = jnp.max(lg - fm_pair * 1e30, axis=1, keepdims=True)
    e = jnp.exp(jnp.minimum(lg - m_all, 80.0))
    s_id = rsum(e * fm_lab_n)
    s_cam = rsum(e * fm_pair_n)
    log_s_id = m_all + jnp.log(s_id)
    log_s_cam = m_all + jnp.log(s_cam)

    lg_diag = jnp.sum(frow * frow, axis=1, keepdims=True)

    cnt_id = rsum(fm_lab) - 1.0
    cnt_cam = rsum(fm_pair) - 1.0

    t_id = rsum(_softplus(log_s_id - lg) * fm_lab)
    t_cam = rsum(_softplus(log_s_cam - lg) * fm_pair)
    row_id = (t_id - _softplus(log_s_id - lg_diag)) / cnt_id
    row_cam = (t_cam - _softplus(log_s_cam - lg_diag)) / cnt_cam

    blk_sum = (jnp.sum(row_id + 0.5 * row_cam) * (1.0 / _M)).reshape(1, 1)

    @pl.when(i == 0)
    def _():
        out_ref[...] = jnp.zeros_like(out_ref)

    out_ref[...] += blk_sum


def kernel(features, label, camid):
    f = features.reshape(_M, _D) * jnp.sqrt(jnp.float32(1.0 / _TEMP))
    lab2 = jnp.repeat(label, 2)
    key2 = lab2 * 8 + jnp.repeat(camid, 2)
    lab_all = lab2.reshape(1, _M)
    key_all = key2.reshape(1, _M)
    lab_col = lab2.reshape(_M, 1)
    key_col = key2.reshape(_M, 1)
    ones = jnp.ones((_M, 1), jnp.float32)

    grid = _M // _BLK
    out = pl.pallas_call(
        _scl_kernel,
        grid=(grid,),
        in_specs=[
            pl.BlockSpec((1, _M), lambda i: (0, 0)),
            pl.BlockSpec((1, _M), lambda i: (0, 0)),
            pl.BlockSpec((_BLK, 1), lambda i: (i, 0)),
            pl.BlockSpec((_BLK, 1), lambda i: (i, 0)),
            pl.BlockSpec((_BLK, _D), lambda i: (i, 0)),
            pl.BlockSpec((_M, _D), lambda i: (0, 0)),
            pl.BlockSpec((_M, 1), lambda i: (0, 0)),
        ],
        out_specs=pl.BlockSpec((1, 1), lambda i: (0, 0)),
        out_shape=jax.ShapeDtypeStruct((1, 1), jnp.float32),
    )(lab_all, key_all, lab_col, key_col, f, f, ones)
    return out.reshape(())


# speedup vs baseline: 1.1089x; 1.0126x over previous
import jax
import jax.numpy as jnp
from jax.experimental import pallas as pl

_TEMP = 0.1
_N = 1024
_M = 2 * _N
_D = 256
_BLK = 512


def _softplus(x):
    return jnp.where(x > 80.0, x, jnp.log1p(jnp.exp(x)))


def _scl_kernel(lab_all_ref, key_all_ref, lab_col_ref, key_col_ref,
                fall_ref, ones_ref, out_ref):
    i = pl.program_id(0)
    frow = fall_ref[pl.ds(i * _BLK, _BLK), :] * (1.0 / _TEMP)
    ones = ones_ref[...]

    def rsum(y):
        return jax.lax.dot_general(
            y, ones, dimension_numbers=(((1,), (0,)), ((), ())),
            preferred_element_type=jnp.float32)

    lg = jax.lax.dot_general(
        frow, fall_ref[...],
        dimension_numbers=(((1,), (1,)), ((), ())),
        preferred_element_type=jnp.float32)

    fm_lab = (lab_col_ref[...] == lab_all_ref[...]).astype(jnp.float32)
    fm_pair = (key_col_ref[...] == key_all_ref[...]).astype(jnp.float32)
    fm_lab_n = 1.0 - fm_lab
    fm_pair_n = 1.0 - fm_pair

    m_all = jnp.max(lg - fm_pair * 1e30, axis=1, keepdims=True)
    e = jnp.exp(jnp.minimum(lg - m_all, 80.0))
    s_id = rsum(e * fm_lab_n)
    s_cam = rsum(e * fm_pair_n)
    log_s_id = m_all + jnp.log(s_id)
    log_s_cam = m_all + jnp.log(s_cam)

    lg_diag = jnp.sum(frow * frow, axis=1, keepdims=True) * _TEMP

    cnt_id = rsum(fm_lab) - 1.0
    cnt_cam = rsum(fm_pair) - 1.0

    t_id = rsum(_softplus(log_s_id - lg) * fm_lab)
    t_cam = rsum(_softplus(log_s_cam - lg) * fm_pair)
    row_id = (t_id - _softplus(log_s_id - lg_diag)) / cnt_id
    row_cam = (t_cam - _softplus(log_s_cam - lg_diag)) / cnt_cam

    blk_sum = (jnp.sum(row_id + 0.5 * row_cam) * (1.0 / _M)).reshape(1, 1)

    @pl.when(i == 0)
    def _():
        out_ref[...] = jnp.zeros_like(out_ref)

    out_ref[...] += blk_sum


def kernel(features, label, camid):
    f = features.reshape(_M, _D)
    lab2 = jnp.repeat(label, 2)
    key2 = lab2 * 8 + jnp.repeat(camid, 2)
    lab_all = lab2.reshape(1, _M)
    key_all = key2.reshape(1, _M)
    lab_col = lab2.reshape(_M, 1)
    key_col = key2.reshape(_M, 1)
    ones = jnp.ones((_M, 1), jnp.float32)

    grid = _M // _BLK
    out = pl.pallas_call(
        _scl_kernel,
        grid=(grid,),
        in_specs=[
            pl.BlockSpec((1, _M), lambda i: (0, 0)),
            pl.BlockSpec((1, _M), lambda i: (0, 0)),
            pl.BlockSpec((_BLK, 1), lambda i: (i, 0)),
            pl.BlockSpec((_BLK, 1), lambda i: (i, 0)),
            pl.BlockSpec((_M, _D), lambda i: (0, 0)),
            pl.BlockSpec((_M, 1), lambda i: (0, 0)),
        ],
        out_specs=pl.BlockSpec((1, 1), lambda i: (0, 0)),
        out_shape=jax.ShapeDtypeStruct((1, 1), jnp.float32),
    )(lab_all, key_all, lab_col, key_col, f, ones)
    return out.reshape(())


# speedup vs baseline: 1.1274x; 1.0167x over previous
import jax
import jax.numpy as jnp
from jax.experimental import pallas as pl

_TEMP = 0.1
_N = 1024
_M = 2 * _N
_D = 256
_BLK = 1024


def _softplus(x):
    return jnp.where(x > 80.0, x, jnp.log1p(jnp.exp(x)))


def _scl_kernel(lab_all_ref, key_all_ref, lab_col_ref, key_col_ref,
                fall_ref, ones_ref, out_ref):
    i = pl.program_id(0)
    frow = fall_ref[pl.ds(i * _BLK, _BLK), :] * (1.0 / _TEMP)
    ones = ones_ref[...]

    def rsum(y):
        return jax.lax.dot_general(
            y, ones, dimension_numbers=(((1,), (0,)), ((), ())),
            preferred_element_type=jnp.float32)

    lg = jax.lax.dot_general(
        frow, fall_ref[...],
        dimension_numbers=(((1,), (1,)), ((), ())),
        preferred_element_type=jnp.float32)

    fm_lab = (lab_col_ref[...] == lab_all_ref[...]).astype(jnp.float32)
    fm_pair = (key_col_ref[...] == key_all_ref[...]).astype(jnp.float32)
    fm_lab_n = 1.0 - fm_lab
    fm_pair_n = 1.0 - fm_pair

    m_all = jnp.max(lg - fm_pair * 1e30, axis=1, keepdims=True)
    e = jnp.exp(jnp.minimum(lg - m_all, 80.0))
    s_id = rsum(e * fm_lab_n)
    s_cam = rsum(e * fm_pair_n)
    log_s_id = m_all + jnp.log(s_id)
    log_s_cam = m_all + jnp.log(s_cam)

    lg_diag = jnp.sum(frow * frow, axis=1, keepdims=True) * _TEMP

    cnt_id = rsum(fm_lab) - 1.0
    cnt_cam = rsum(fm_pair) - 1.0

    t_id = rsum(_softplus(log_s_id - lg) * fm_lab)
    t_cam = rsum(_softplus(log_s_cam - lg) * fm_pair)
    row_id = (t_id - _softplus(log_s_id - lg_diag)) / cnt_id
    row_cam = (t_cam - _softplus(log_s_cam - lg_diag)) / cnt_cam

    blk_sum = (jnp.sum(row_id + 0.5 * row_cam) * (1.0 / _M)).reshape(1, 1)

    @pl.when(i == 0)
    def _():
        out_ref[...] = jnp.zeros_like(out_ref)

    out_ref[...] += blk_sum


def kernel(features, label, camid):
    f = features.reshape(_M, _D)
    lab2 = jnp.repeat(label, 2)
    key2 = lab2 * 8 + jnp.repeat(camid, 2)
    lab_all = lab2.reshape(1, _M)
    key_all = key2.reshape(1, _M)
    lab_col = lab2.reshape(_M, 1)
    key_col = key2.reshape(_M, 1)
    ones = jnp.ones((_M, 1), jnp.float32)

    grid = _M // _BLK
    out = pl.pallas_call(
        _scl_kernel,
        grid=(grid,),
        in_specs=[
            pl.BlockSpec((1, _M), lambda i: (0, 0)),
            pl.BlockSpec((1, _M), lambda i: (0, 0)),
            pl.BlockSpec((_BLK, 1), lambda i: (i, 0)),
            pl.BlockSpec((_BLK, 1), lambda i: (i, 0)),
            pl.BlockSpec((_M, _D), lambda i: (0, 0)),
            pl.BlockSpec((_M, 1), lambda i: (0, 0)),
        ],
        out_specs=pl.BlockSpec((1, 1), lambda i: (0, 0)),
        out_shape=jax.ShapeDtypeStruct((1, 1), jnp.float32),
    )(lab_all, key_all, lab_col, key_col, f, ones)
    return out.reshape(())
